# baseline (speedup 1.0000x reference)
import functools

import jax
import jax.numpy as jnp
from jax import lax
from jax.experimental import pallas as pl
from jax.experimental.pallas import tpu as pltpu
from jax.experimental.pallas import tpu_sc as plsc

_NC = 2
_NS = 16
_LANES = 16
_RC = 16
_NBUF = 4


def _sc_add(B, T, C, x_hbm, pe_hbm, out_hbm,
            x0, x1, x2, x3, pe_v, l0, l1, l2, l3, s0, s1, s2, s3):
    nw = _NC * _NS
    wid = lax.axis_index("s") * _NC + lax.axis_index("c")
    rows_per_w = T // nw
    t0 = wid * rows_per_w
    n_j = (rows_per_w // _RC) * B

    bufs = (x0, x1, x2, x3)
    lsems = (l0, l1, l2, l3)
    ssems = (s0, s1, s2, s3)

    def row0(j):
        return t0 + (j // B) * _RC

    def load(j):
        return pltpu.make_async_copy(
            x_hbm.at[j % B, pl.ds(row0(j), _RC)], bufs[j % _NBUF],
            lsems[j % _NBUF])

    def store(j):
        return pltpu.make_async_copy(
            bufs[j % _NBUF], out_hbm.at[j % B, pl.ds(row0(j), _RC)],
            ssems[j % _NBUF])

    for j in range(_NBUF - 1):
        load(j).start()
    for j in range(n_j):
        xb = bufs[j % _NBUF]
        if j % B == 0:
            pltpu.sync_copy(pe_hbm.at[pl.ds(row0(j), _RC)], pe_v)
        load(j).wait()

        unr = 8
        vecs_per_row = C // (_LANES * unr)
        def add_body(i, _, xb=xb):
            r = i // vecs_per_row
            base = (i % vecs_per_row) * (_LANES * unr)
            for k in range(unr):
                s = base + k * _LANES
                plsc.addupdate(xb.at[r, pl.ds(s, _LANES)],
                               pe_v[r, pl.ds(s, _LANES)])
            return _
        lax.fori_loop(0, _RC * vecs_per_row, add_body, 0)

        store(j).start()
        nxt = j + _NBUF - 1
        if nxt < n_j:
            if j >= 1:
                store(j - 1).wait()
            load(nxt).start()
    for m in range(max(0, n_j - _NBUF), n_j):
        store(m).wait()


def kernel(x, pos_emb):
    B, T, C = x.shape
    mesh = plsc.VectorSubcoreMesh(core_axis_name="c", subcore_axis_name="s")
    f32 = jnp.float32
    run = pl.kernel(
        functools.partial(_sc_add, B, T, C),
        out_type=jax.ShapeDtypeStruct((B, T, C), f32),
        mesh=mesh,
        scratch_types=(
            [pltpu.VMEM((_RC, C), f32)] * (_NBUF + 1)
            + [pltpu.SemaphoreType.DMA] * (2 * _NBUF)
        ),
    )
    return run(x, pos_emb)


# speedup vs baseline: 2.1586x; 2.1586x over previous
import functools

import jax
import jax.numpy as jnp
from jax import lax
from jax.experimental import pallas as pl
from jax.experimental.pallas import tpu as pltpu
from jax.experimental.pallas import tpu_sc as plsc

_NC = 2
_NS = 16
_LANES = 16
_RC = 8
_RING = 3


def _sc_add(B, T, C, x_hbm, pe_hbm, out_hbm, *scr):
    nw = _NC * _NS
    wid = lax.axis_index("s") * _NC + lax.axis_index("c")
    rows_per_w = T // nw
    t0 = wid * rows_per_w
    n_c = rows_per_w // _RC

    nb = B * _RING
    xbufs = [[scr[b * _RING + p] for p in range(_RING)] for b in range(B)]
    pebufs = (scr[nb], scr[nb + 1])
    lsems = [[scr[nb + 2 + b * _RING + p] for p in range(_RING)]
             for b in range(B)]
    ssems = [[scr[2 * nb + 2 + b * _RING + p] for p in range(_RING)]
             for b in range(B)]
    pesems = (scr[3 * nb + 2], scr[3 * nb + 3])

    def row0(c):
        return t0 + c * _RC

    def load(c, b):
        p = c % _RING
        return pltpu.make_async_copy(
            x_hbm.at[b, pl.ds(row0(c), _RC)], xbufs[b][p], lsems[b][p])

    def store(c, b):
        p = c % _RING
        return pltpu.make_async_copy(
            xbufs[b][p], out_hbm.at[b, pl.ds(row0(c), _RC)], ssems[b][p])

    def pe_load(c):
        return pltpu.make_async_copy(
            pe_hbm.at[pl.ds(row0(c), _RC)], pebufs[c % 2], pesems[c % 2])

    pe_load(0).start()
    pe_load(1).start()
    for c in range(_RING - 1):
        for b in range(B):
            load(c, b).start()

    gpr = C // _LANES
    n_i = _RC * gpr // 2

    for c in range(n_c):
        pe_v = pebufs[c % 2]
        pe_load(c).wait()
        for b in range(B):
            load(c, b).wait()

        bufs_c = [xbufs[b][c % _RING] for b in range(B)]

        def add_body(i, _, bufs_c=bufs_c, pe_v=pe_v):
            for k in range(2):
                g = i * 2 + k
                r = g // gpr
                s = (g % gpr) * _LANES
                pv = pe_v[r, pl.ds(s, _LANES)]
                for xb in bufs_c:
                    xb[r, pl.ds(s, _LANES)] = xb[r, pl.ds(s, _LANES)] + pv
            return _
        lax.fori_loop(0, n_i, add_body, 0)

        for b in range(B):
            store(c, b).start()
        if c + 2 < n_c:
            pe_load(c + 2).start()
        if c + _RING - 1 < n_c:
            if c >= 1:
                for b in range(B):
                    store(c - 1, b).wait()
            for b in range(B):
                load(c + _RING - 1, b).start()
    for m in range(max(0, n_c - _RING), n_c):
        for b in range(B):
            store(m, b).wait()


def kernel(x, pos_emb):
    B, T, C = x.shape
    mesh = plsc.VectorSubcoreMesh(core_axis_name="c", subcore_axis_name="s")
    f32 = jnp.float32
    nb = B * _RING
    run = pl.kernel(
        functools.partial(_sc_add, B, T, C),
        out_type=jax.ShapeDtypeStruct((B, T, C), f32),
        mesh=mesh,
        scratch_types=(
            [pltpu.VMEM((_RC, C), f32)] * (nb + 2)
            + [pltpu.SemaphoreType.DMA] * (2 * nb + 2)
        ),
    )
    return run(x, pos_emb)
